# Initial kernel scaffold; baseline (speedup 1.0000x reference)
import jax
import jax.numpy as jnp
from jax.experimental import pallas as pl


def kernel(x, edge_index, batch, W0_1, b0_1, W0_2, b0_2, W1_1, b1_1, W1_2, b1_2, WL_1, bL_1, WL_2, bL_2):
    raise NotImplementedError("write your pallas kernel here")


# speedup vs baseline: 4.6984x; 4.6984x over previous
import functools

import jax
import jax.numpy as jnp
from jax import lax
from jax.experimental import pallas as pl
from jax.experimental.pallas import tpu as pltpu
from jax.experimental.pallas import tpu_sc as plsc

N = 10000
E = 320000
D = 128
NUM_GRAPHS = 64

_NW = 32
_EDGES_PER_TILE = E // _NW
_K = 80
_ITERS = _EDGES_PER_TILE // _K
_RPT = 624
_RPT_LAST = N - 15 * _RPT


def _sc_agg_body(x_hbm, src_hbm, dst_hbm, out_hbm, idx_s, idx_d, rows, acc, sem):
    c = lax.axis_index("c")
    s = lax.axis_index("s")
    w = c * 16 + s
    r0 = s * _RPT

    @pl.when(s < 15)
    def _():
        pltpu.sync_copy(x_hbm.at[pl.ds(r0, _RPT)], acc.at[pl.ds(r0, _RPT)])

    @pl.when(s == 15)
    def _():
        pltpu.sync_copy(x_hbm.at[pl.ds(15 * _RPT, _RPT_LAST)],
                        acc.at[pl.ds(15 * _RPT, _RPT_LAST)])

    plsc.subcore_barrier()

    base = w * _EDGES_PER_TILE

    def body(i, carry):
        off = base + i * _K
        pltpu.sync_copy(src_hbm.at[pl.ds(off, _K)], idx_s)
        pltpu.sync_copy(dst_hbm.at[pl.ds(off, _K)], idx_d)
        pltpu.async_copy(x_hbm.at[idx_s], rows, sem).wait()
        pltpu.sync_copy(rows, acc.at[idx_d], add=True)
        return carry

    lax.fori_loop(0, _ITERS, body, 0)
    plsc.subcore_barrier()

    @pl.when(s < 15)
    def _():
        pltpu.sync_copy(acc.at[pl.ds(r0, _RPT)],
                        out_hbm.at[c, pl.ds(r0, _RPT)])

    @pl.when(s == 15)
    def _():
        pltpu.sync_copy(acc.at[pl.ds(15 * _RPT, _RPT_LAST)],
                        out_hbm.at[c, pl.ds(15 * _RPT, _RPT_LAST)])


_sc_agg = functools.partial(
    pl.kernel,
    out_type=jax.ShapeDtypeStruct((2, N, D), jnp.float32),
    mesh=plsc.VectorSubcoreMesh(core_axis_name="c", subcore_axis_name="s"),
    scratch_types=[
        pltpu.VMEM((_K,), jnp.int32),
        pltpu.VMEM((_K,), jnp.int32),
        pltpu.VMEM((_K, D), jnp.float32),
        pltpu.VMEM_SHARED((N, D), jnp.float32),
        pltpu.SemaphoreType.DMA,
    ],
)(_sc_agg_body)


_BM = 1000


def _mlp_kernel(p_ref, q_ref, x_ref, w1_ref, b1_ref, w2_ref, b2_ref, o_ref):
    h = p_ref[0] + q_ref[0] - x_ref[...]
    t = jnp.maximum(
        jnp.dot(h, w1_ref[...], preferred_element_type=jnp.float32) + b1_ref[...],
        0.0)
    o_ref[...] = (
        jnp.dot(t, w2_ref[...], preferred_element_type=jnp.float32) + b2_ref[...])


def _conv_mlp(parts, x, w1, b1, w2, b2):
    return pl.pallas_call(
        _mlp_kernel,
        grid=(N // _BM,),
        in_specs=[
            pl.BlockSpec((1, _BM, D), lambda i: (0, i, 0)),
            pl.BlockSpec((1, _BM, D), lambda i: (1, i, 0)),
            pl.BlockSpec((_BM, D), lambda i: (i, 0)),
            pl.BlockSpec((D, D), lambda i: (0, 0)),
            pl.BlockSpec((1, D), lambda i: (0, 0)),
            pl.BlockSpec((D, D), lambda i: (0, 0)),
            pl.BlockSpec((1, D), lambda i: (0, 0)),
        ],
        out_specs=pl.BlockSpec((_BM, D), lambda i: (i, 0)),
        out_shape=jax.ShapeDtypeStruct((N, D), jnp.float32),
    )(parts, parts, x, w1, b1.reshape(1, D), w2, b2.reshape(1, D))


_BP = 400
_NP = N // _BP


def _pool_kernel(b_ref, x1_ref, x2_ref, wa_ref, wb_ref, bl1_ref, wl2_ref,
                 bl2_ref, o_ref, acc1, acc2):
    i = pl.program_id(0)
    seg = b_ref[0, 0, :]
    iota = lax.broadcasted_iota(jnp.int32, (NUM_GRAPHS, _BP), 0)
    onehot = (seg[None, :] == iota).astype(jnp.float32)
    dn = (((1,), (0,)), ((), ()))
    d1 = lax.dot_general(onehot, x1_ref[...], dn,
                         preferred_element_type=jnp.float32)
    d2 = lax.dot_general(onehot, x2_ref[...], dn,
                         preferred_element_type=jnp.float32)

    @pl.when(i == 0)
    def _():
        acc1[...] = d1
        acc2[...] = d2

    @pl.when(i > 0)
    def _():
        acc1[...] += d1
        acc2[...] += d2

    @pl.when(i == _NP - 1)
    def _():
        h = jnp.maximum(
            jnp.dot(acc1[...], wa_ref[...], preferred_element_type=jnp.float32)
            + jnp.dot(acc2[...], wb_ref[...], preferred_element_type=jnp.float32)
            + bl1_ref[...], 0.0)
        o_ref[...] = (
            jnp.dot(h, wl2_ref[...], preferred_element_type=jnp.float32)
            + bl2_ref[...])


def _pool_final(batch, x1, x2, wl1, bl1, wl2, bl2):
    return pl.pallas_call(
        _pool_kernel,
        grid=(_NP,),
        in_specs=[
            pl.BlockSpec((1, 1, _BP), lambda i: (i, 0, 0)),
            pl.BlockSpec((_BP, D), lambda i: (i, 0)),
            pl.BlockSpec((_BP, D), lambda i: (i, 0)),
            pl.BlockSpec((D, D), lambda i: (0, 0)),
            pl.BlockSpec((D, D), lambda i: (0, 0)),
            pl.BlockSpec((1, D), lambda i: (0, 0)),
            pl.BlockSpec((D, D), lambda i: (0, 0)),
            pl.BlockSpec((1, D), lambda i: (0, 0)),
        ],
        out_specs=pl.BlockSpec((NUM_GRAPHS, D), lambda i: (0, 0)),
        out_shape=jax.ShapeDtypeStruct((NUM_GRAPHS, D), jnp.float32),
        scratch_shapes=[
            pltpu.VMEM((NUM_GRAPHS, D), jnp.float32),
            pltpu.VMEM((NUM_GRAPHS, D), jnp.float32),
        ],
    )(batch.reshape(_NP, 1, _BP), x1, x2, wl1[:D], wl1[D:],
      bl1.reshape(1, D), wl2, bl2.reshape(1, D))


def kernel(x, edge_index, batch,
           W0_1, b0_1, W0_2, b0_2,
           W1_1, b1_1, W1_2, b1_2,
           WL_1, bL_1, WL_2, bL_2):
    src = edge_index[0]
    dst = edge_index[1]
    p = _sc_agg(x, src, dst)
    x1 = _conv_mlp(p, x, W0_1, b0_1, W0_2, b0_2)
    p2 = _sc_agg(x1, src, dst)
    x2 = _conv_mlp(p2, x1, W1_1, b1_1, W1_2, b1_2)
    return _pool_final(batch, x1, x2, WL_1, bL_1, WL_2, bL_2)


# speedup vs baseline: 12.3415x; 2.6267x over previous
import functools

import jax
import jax.numpy as jnp
from jax import lax
from jax.experimental import pallas as pl
from jax.experimental.pallas import tpu as pltpu
from jax.experimental.pallas import tpu_sc as plsc

N = 10000
E = 320000
D = 128
NUM_GRAPHS = 64

_NW = 32
_EDGES_PER_TILE = E // _NW
_K = 80
_ITERS = _EDGES_PER_TILE // _K
_NBUF = 3
_FULL = _ITERS // _NBUF
_TAIL = _ITERS - _FULL * _NBUF
_RPT = 624
_RPT_LAST = N - 15 * _RPT


def _sc_agg_body(x_hbm, src_hbm, dst_hbm, out_hbm, sidx, rows,
                 d0, d1, d2, acc, *sems):
    c = lax.axis_index("c")
    s = lax.axis_index("s")
    w = c * 16 + s
    r0 = s * _RPT
    dbufs = (d0, d1, d2)
    semg = sems[:_NBUF]
    semi = sems[_NBUF:]

    @pl.when(s < 15)
    def _():
        pltpu.sync_copy(x_hbm.at[pl.ds(r0, _RPT)], acc.at[pl.ds(r0, _RPT)])

    @pl.when(s == 15)
    def _():
        pltpu.sync_copy(x_hbm.at[pl.ds(15 * _RPT, _RPT_LAST)],
                        acc.at[pl.ds(15 * _RPT, _RPT_LAST)])

    base = w * _EDGES_PER_TILE
    pltpu.sync_copy(src_hbm.at[pl.ds(base, _EDGES_PER_TILE)], sidx)

    def issue(chunk, b):
        pltpu.async_copy(dst_hbm.at[pl.ds(base + chunk * _K, _K)],
                         dbufs[b], semi[b])
        pltpu.async_copy(x_hbm.at[sidx.at[pl.ds(chunk * _K, _K)]],
                         rows.at[b], semg[b])

    for b in range(_NBUF):
        issue(b, b)

    plsc.subcore_barrier()

    def drain_and_scatter(b):
        pltpu.make_async_copy(dst_hbm.at[pl.ds(0, _K)], dbufs[b],
                              semi[b]).wait()
        pltpu.make_async_copy(x_hbm.at[sidx.at[pl.ds(0, _K)]],
                              rows.at[b], semg[b]).wait()
        pltpu.sync_copy(rows.at[b], acc.at[dbufs[b]], add=True)

    def body(i, carry):
        for b in range(_NBUF):
            chunk = i * _NBUF + b
            drain_and_scatter(b)

            @pl.when(chunk + _NBUF < _ITERS)
            def _():
                issue(chunk + _NBUF, b)
        return carry

    lax.fori_loop(0, _FULL, body, 0)
    for b in range(_TAIL):
        drain_and_scatter(b)
    plsc.subcore_barrier()

    @pl.when(s < 15)
    def _():
        pltpu.sync_copy(acc.at[pl.ds(r0, _RPT)],
                        out_hbm.at[c, pl.ds(r0, _RPT)])

    @pl.when(s == 15)
    def _():
        pltpu.sync_copy(acc.at[pl.ds(15 * _RPT, _RPT_LAST)],
                        out_hbm.at[c, pl.ds(15 * _RPT, _RPT_LAST)])


_sc_agg = functools.partial(
    pl.kernel,
    out_type=jax.ShapeDtypeStruct((2, N, D), jnp.float32),
    mesh=plsc.VectorSubcoreMesh(core_axis_name="c", subcore_axis_name="s"),
    scratch_types=[
        pltpu.VMEM((_EDGES_PER_TILE,), jnp.int32),
        pltpu.VMEM((_NBUF, _K, D), jnp.float32),
        pltpu.VMEM((_K,), jnp.int32),
        pltpu.VMEM((_K,), jnp.int32),
        pltpu.VMEM((_K,), jnp.int32),
    ] + [
        pltpu.VMEM_SHARED((N, D), jnp.float32),
    ] + [pltpu.SemaphoreType.DMA] * (2 * _NBUF),
)(_sc_agg_body)


_BM = 1000


def _mlp_kernel(p_ref, q_ref, x_ref, w1_ref, b1_ref, w2_ref, b2_ref, o_ref):
    h = p_ref[0] + q_ref[0] - x_ref[...]
    t = jnp.maximum(
        jnp.dot(h, w1_ref[...], preferred_element_type=jnp.float32) + b1_ref[...],
        0.0)
    o_ref[...] = (
        jnp.dot(t, w2_ref[...], preferred_element_type=jnp.float32) + b2_ref[...])


def _conv_mlp(parts, x, w1, b1, w2, b2):
    return pl.pallas_call(
        _mlp_kernel,
        grid=(N // _BM,),
        in_specs=[
            pl.BlockSpec((1, _BM, D), lambda i: (0, i, 0)),
            pl.BlockSpec((1, _BM, D), lambda i: (1, i, 0)),
            pl.BlockSpec((_BM, D), lambda i: (i, 0)),
            pl.BlockSpec((D, D), lambda i: (0, 0)),
            pl.BlockSpec((1, D), lambda i: (0, 0)),
            pl.BlockSpec((D, D), lambda i: (0, 0)),
            pl.BlockSpec((1, D), lambda i: (0, 0)),
        ],
        out_specs=pl.BlockSpec((_BM, D), lambda i: (i, 0)),
        out_shape=jax.ShapeDtypeStruct((N, D), jnp.float32),
    )(parts, parts, x, w1, b1.reshape(1, D), w2, b2.reshape(1, D))


_BP = 400
_NP = N // _BP


def _pool_kernel(b_ref, x1_ref, x2_ref, wa_ref, wb_ref, bl1_ref, wl2_ref,
                 bl2_ref, o_ref, acc1, acc2):
    i = pl.program_id(0)
    seg = b_ref[0, 0, :]
    iota = lax.broadcasted_iota(jnp.int32, (NUM_GRAPHS, _BP), 0)
    onehot = (seg[None, :] == iota).astype(jnp.float32)
    dn = (((1,), (0,)), ((), ()))
    d1 = lax.dot_general(onehot, x1_ref[...], dn,
                         preferred_element_type=jnp.float32)
    d2 = lax.dot_general(onehot, x2_ref[...], dn,
                         preferred_element_type=jnp.float32)

    @pl.when(i == 0)
    def _():
        acc1[...] = d1
        acc2[...] = d2

    @pl.when(i > 0)
    def _():
        acc1[...] += d1
        acc2[...] += d2

    @pl.when(i == _NP - 1)
    def _():
        h = jnp.maximum(
            jnp.dot(acc1[...], wa_ref[...], preferred_element_type=jnp.float32)
            + jnp.dot(acc2[...], wb_ref[...], preferred_element_type=jnp.float32)
            + bl1_ref[...], 0.0)
        o_ref[...] = (
            jnp.dot(h, wl2_ref[...], preferred_element_type=jnp.float32)
            + bl2_ref[...])


def _pool_final(batch, x1, x2, wl1, bl1, wl2, bl2):
    return pl.pallas_call(
        _pool_kernel,
        grid=(_NP,),
        in_specs=[
            pl.BlockSpec((1, 1, _BP), lambda i: (i, 0, 0)),
            pl.BlockSpec((_BP, D), lambda i: (i, 0)),
            pl.BlockSpec((_BP, D), lambda i: (i, 0)),
            pl.BlockSpec((D, D), lambda i: (0, 0)),
            pl.BlockSpec((D, D), lambda i: (0, 0)),
            pl.BlockSpec((1, D), lambda i: (0, 0)),
            pl.BlockSpec((D, D), lambda i: (0, 0)),
            pl.BlockSpec((1, D), lambda i: (0, 0)),
        ],
        out_specs=pl.BlockSpec((NUM_GRAPHS, D), lambda i: (0, 0)),
        out_shape=jax.ShapeDtypeStruct((NUM_GRAPHS, D), jnp.float32),
        scratch_shapes=[
            pltpu.VMEM((NUM_GRAPHS, D), jnp.float32),
            pltpu.VMEM((NUM_GRAPHS, D), jnp.float32),
        ],
    )(batch.reshape(_NP, 1, _BP), x1, x2, wl1[:D], wl1[D:],
      bl1.reshape(1, D), wl2, bl2.reshape(1, D))


def kernel(x, edge_index, batch,
           W0_1, b0_1, W0_2, b0_2,
           W1_1, b1_1, W1_2, b1_2,
           WL_1, bL_1, WL_2, bL_2):
    src = edge_index[0]
    dst = edge_index[1]
    p = _sc_agg(x, src, dst)
    x1 = _conv_mlp(p, x, W0_1, b0_1, W0_2, b0_2)
    p2 = _sc_agg(x1, src, dst)
    x2 = _conv_mlp(p2, x1, W1_1, b1_1, W1_2, b1_2)
    return _pool_final(batch, x1, x2, WL_1, bL_1, WL_2, bL_2)
